# baseline (speedup 1.0000x reference)
import jax
import jax.numpy as jnp
from jax.experimental import pallas as pl
from jax.experimental.pallas import tpu as pltpu

B, N, L, D = 4, 4096, 12, 64
LD = L * D
BD = B * D
BM = 512

_bf16 = jnp.bfloat16
_f32 = jnp.float32


def _mlp_in_kernel(x_ref, w_ref, b_ref, o_ref):
    w = w_ref[:].astype(_bf16)
    bias = b_ref[:]
    for bi in range(B):
        xb = x_ref[bi].astype(_bf16)
        h = jnp.dot(xb, w, preferred_element_type=_f32) + bias
        o_ref[:, bi * D:(bi + 1) * D] = jnp.maximum(h, 0.0).astype(_bf16)


def _gcn_kernel(a_ref, h_ref, w1_ref, b1_ref, w2_ref, b2_ref, w3_ref, b3_ref,
                o_ref, z_ref, h2_ref):
    RC = 1024
    layers = ((w1_ref, b1_ref), (w2_ref, b2_ref), (w3_ref, b3_ref))
    src = h_ref
    for li, (w_ref, b_ref) in enumerate(layers):
        w = w_ref[:].astype(_bf16)
        for bi in range(B):
            sl = slice(bi * D, (bi + 1) * D)
            z_ref[:, sl] = jnp.dot(src[:, sl], w,
                                   preferred_element_type=_f32).astype(_bf16)
        dst = o_ref if li == 2 else h2_ref
        for rc in range(N // RC):
            rs = slice(rc * RC, (rc + 1) * RC)
            agg = jnp.dot(a_ref[rs], z_ref[:], preferred_element_type=_f32)
            dst[rs] = jnp.maximum(agg + b_ref[:], 0.0).astype(_bf16)
        src = h2_ref


def _mlp_out_kernel(h_ref, w_ref, b_ref, o_ref):
    w = w_ref[:].astype(_bf16)
    bias = b_ref[:]
    for bi in range(B):
        hb = h_ref[:, bi * D:(bi + 1) * D]
        o = jnp.dot(hb, w, preferred_element_type=_f32) + bias
        o_ref[bi] = jnp.maximum(o, 0.0)


def kernel(x, adj, W_mlp2, b_mlp2, W_g1, b_g1, W_g2, b_g2, W_g3, b_g3,
           W_mlp1, b_mlp1):
    xf = x.reshape(B, N, LD)
    adj_bf = adj.astype(_bf16)
    b2 = b_mlp2.reshape(1, D)
    bt = [jnp.tile(b, B).reshape(1, BD) for b in (b_g1, b_g2, b_g3)]
    b1 = b_mlp1.reshape(1, LD)

    h0 = pl.pallas_call(
        _mlp_in_kernel,
        grid=(N // BM,),
        in_specs=[
            pl.BlockSpec((B, BM, LD), lambda i: (0, i, 0)),
            pl.BlockSpec((LD, D), lambda i: (0, 0)),
            pl.BlockSpec((1, D), lambda i: (0, 0)),
        ],
        out_specs=pl.BlockSpec((BM, BD), lambda i: (i, 0)),
        out_shape=jax.ShapeDtypeStruct((N, BD), _bf16),
    )(xf, W_mlp2, b2)

    h3 = pl.pallas_call(
        _gcn_kernel,
        in_specs=[pl.BlockSpec(memory_space=pltpu.VMEM)] * 8,
        out_specs=pl.BlockSpec(memory_space=pltpu.VMEM),
        out_shape=jax.ShapeDtypeStruct((N, BD), _bf16),
        scratch_shapes=[pltpu.VMEM((N, BD), _bf16),
                        pltpu.VMEM((N, BD), _bf16)],
    )(adj_bf, h0, W_g1, bt[0], W_g2, bt[1], W_g3, bt[2])

    out = pl.pallas_call(
        _mlp_out_kernel,
        grid=(N // BM,),
        in_specs=[
            pl.BlockSpec((BM, BD), lambda i: (i, 0)),
            pl.BlockSpec((D, LD), lambda i: (0, 0)),
            pl.BlockSpec((1, LD), lambda i: (0, 0)),
        ],
        out_specs=pl.BlockSpec((B, BM, LD), lambda i: (0, i, 0)),
        out_shape=jax.ShapeDtypeStruct((B, N, LD), _f32),
    )(h3, W_mlp1, b1)
    return out
